# Initial kernel scaffold; baseline (speedup 1.0000x reference)
import jax
import jax.numpy as jnp
from jax.experimental import pallas as pl


def kernel(context_index, context_x, context_y, target_index, target_x, target_y, We1, be1, We2, be2, Wm, bm, Wms, bms, Ws, bs, Wpm, bpm, Wps, bps, Wd1, bd1, Wd2, bd2):
    raise NotImplementedError("write your pallas kernel here")


# speedup vs baseline: 3.6904x; 3.6904x over previous
import jax
import jax.numpy as jnp
from jax import lax
from jax.experimental import pallas as pl
from jax.experimental.pallas import tpu as pltpu

F32 = jnp.float32
EPS = 0.01
BN = 2048


def _onehot(idx, g, dtype):
    iota = lax.broadcasted_iota(jnp.int32, (idx.shape[0], g), 1)
    return (iota == idx[:, None]).astype(dtype)


def _softplus(x):
    return jnp.maximum(x, 0.0) + jnp.log1p(jnp.exp(-jnp.abs(x)))


BF16 = jnp.bfloat16


def _split(b):
    hi = b.astype(BF16)
    lo = (b - hi.astype(F32)).astype(BF16)
    return hi, lo


def _seg_dot(oh, b):
    dn = (((0,), (0,)), ((), ()))
    hi, lo = _split(b)
    return (lax.dot_general(oh, hi, dn, preferred_element_type=F32) +
            lax.dot_general(oh, lo, dn, preferred_element_type=F32))


def _dot3(x, w):
    xh, xl = _split(x)
    wh, wl = _split(w)
    r = jnp.dot(xh, wh, preferred_element_type=F32)
    r += jnp.dot(xh, wl, preferred_element_type=F32)
    r += jnp.dot(xl, wh, preferred_element_type=F32)
    return r


def _gather_dot(oh, t):
    hi, lo = _split(t)
    return (jnp.dot(oh, hi, preferred_element_type=F32) +
            jnp.dot(oh, lo, preferred_element_type=F32))


def _p1_body(g, idx_ref, x_ref, y_ref, w_ref, a_ref, s_ref):
    i = pl.program_id(0)
    a = _dot3(x_ref[...], w_ref[...])
    a_ref[...] = a
    idx = idx_ref[0, 0, :]
    oh = _onehot(idx, g, BF16)
    ones = jnp.ones((a.shape[0], 1), F32)
    b = jnp.concatenate([a, y_ref[...], ones], axis=1)
    contrib = _seg_dot(oh, b)

    @pl.when(i == 0)
    def _():
        s_ref[...] = jnp.zeros_like(s_ref)

    s_ref[...] += contrib


def _run_p1(idx3, x, y, w, g):
    n, d = x.shape
    wout = w.shape[1]
    nb = n // BN
    return pl.pallas_call(
        lambda *a: _p1_body(g, *a),
        grid=(nb,),
        in_specs=[
            pl.BlockSpec((1, 1, BN), lambda i: (i, 0, 0)),
            pl.BlockSpec((BN, d), lambda i: (i, 0)),
            pl.BlockSpec((BN, 1), lambda i: (i, 0)),
            pl.BlockSpec((d, wout), lambda i: (0, 0)),
        ],
        out_specs=[
            pl.BlockSpec((BN, wout), lambda i: (i, 0)),
            pl.BlockSpec((g, wout + 2), lambda i: (0, 0)),
        ],
        out_shape=[
            jax.ShapeDtypeStruct((n, wout), F32),
            jax.ShapeDtypeStruct((g, wout + 2), F32),
        ],
        compiler_params=pltpu.CompilerParams(
            dimension_semantics=("arbitrary",)),
    )(idx3, x, y, w)


def _p2_body(g, h, has_dec, *args):
    if has_dec:
        (idx_ref, a_ref, y_ref, s_ref, wy_ref, be1_ref, bd1_ref, wd2_ref,
         bd2_ref, r_ref, pred_ref, tt_ref) = args
    else:
        (idx_ref, a_ref, y_ref, s_ref, wy_ref, be1_ref, r_ref,
         tt_ref) = args
    i = pl.program_id(0)
    wa = a_ref.shape[1]

    @pl.when(i == 0)
    def _():
        s = s_ref[...]
        cnt = jnp.maximum(s[:, wa + 1:wa + 2], 1.0)
        te = s[:, :h] / cnt + (s[:, wa:wa + 1] / cnt) * wy_ref[...] \
            - be1_ref[...]
        if has_dec:
            td = s[:, h:wa] / cnt - bd1_ref[...]
            tt_ref[...] = jnp.concatenate([te, td], axis=1)
        else:
            tt_ref[...] = te
        r_ref[...] = jnp.zeros_like(r_ref)

    idx = idx_ref[0, 0, :]
    oh = _onehot(idx, g, BF16)
    gt = _gather_dot(oh, tt_ref[...])
    a = a_ref[...]
    h1 = jnp.tanh(a[:, :h] + y_ref[...] * wy_ref[...] - gt[:, :h])
    r_ref[...] += _seg_dot(oh, h1)
    if has_dec:
        d1 = jnp.tanh(a[:, h:] - gt[:, h:])
        pred_ref[...] = _dot3(d1, wd2_ref[...]) + bd2_ref[...]


def _run_p2(idx3, a_stash, y, s, wy, be1, g, h, has_dec,
            bd1=None, wd2=None, bd2=None):
    n, wa = a_stash.shape
    nb = n // BN
    in_specs = [
        pl.BlockSpec((1, 1, BN), lambda i: (i, 0, 0)),
        pl.BlockSpec((BN, wa), lambda i: (i, 0)),
        pl.BlockSpec((BN, 1), lambda i: (i, 0)),
        pl.BlockSpec((g, wa + 2), lambda i: (0, 0)),
        pl.BlockSpec((1, h), lambda i: (0, 0)),
        pl.BlockSpec((1, h), lambda i: (0, 0)),
    ]
    outs = [pl.BlockSpec((g, h), lambda i: (0, 0))]
    out_shape = [jax.ShapeDtypeStruct((g, h), F32)]
    operands = [idx3, a_stash, y, s, wy, be1]
    if has_dec:
        kout = wd2.shape[1]
        in_specs += [
            pl.BlockSpec((1, h), lambda i: (0, 0)),
            pl.BlockSpec((h, kout), lambda i: (0, 0)),
            pl.BlockSpec((1, kout), lambda i: (0, 0)),
        ]
        outs.append(pl.BlockSpec((BN, kout), lambda i: (i, 0)))
        out_shape.append(jax.ShapeDtypeStruct((n, kout), F32))
        operands += [bd1, wd2, bd2]
    return pl.pallas_call(
        lambda *args: _p2_body(g, h, has_dec, *args),
        grid=(nb,),
        in_specs=in_specs,
        out_specs=outs,
        out_shape=out_shape,
        scratch_shapes=[pltpu.VMEM((g, wa), F32)],
        compiler_params=pltpu.CompilerParams(
            dimension_semantics=("arbitrary",)),
    )(*operands)


def _p3_body(h, k, s_c_ref, s_t_ref, r_c_ref, r_t_ref, we2_ref, be2_ref,
             wm_ref, bm_ref, wms_ref, bms_ref, ws_ref, bs_ref, wpm_ref,
             bpm_ref, wps_ref, bps_ref,
             prob_ref, rep_ref, kl_ref, scale_ref, tf_ref):
    s_c = s_c_ref[...]
    s_t = s_t_ref[...]
    wc = s_c.shape[1] - 2
    wt = s_t.shape[1] - 2
    cnt_c = jnp.maximum(s_c[:, wc + 1:wc + 2], 1.0)
    cnt_t = jnp.maximum(s_t[:, wt + 1:wt + 2], 1.0)

    def mm(x, w_ref, b_ref):
        return jnp.dot(x, w_ref[...], preferred_element_type=F32,
                       precision=lax.Precision.HIGHEST) + b_ref[...]

    rep_c = mm(r_c_ref[...] / cnt_c, we2_ref, be2_ref)
    rep_t = mm(r_t_ref[...] / cnt_t, we2_ref, be2_ref)

    def heads(rep):
        ml = mm(rep, wm_ref, bm_ref)
        msig = _softplus(mm(rep, wms_ref, bms_ref)) * (1.0 - EPS) + EPS
        pmu = mm(rep, wpm_ref, bpm_ref)
        psig = _softplus(mm(rep, wps_ref, bps_ref)) * (1.0 - EPS) + EPS
        return ml, msig, pmu, psig

    ml_c, msig_c, pmu_c, psig_c = heads(rep_c)
    ml_t, msig_t, pmu_t, psig_t = heads(rep_t)

    mx = jnp.max(ml_t, axis=1, keepdims=True)
    ex = jnp.exp(ml_t - mx)
    prob_t = ex / jnp.sum(ex, axis=1, keepdims=True)
    scale_t = _softplus(mm(rep_t, ws_ref, bs_ref)) * (1.0 - EPS) + EPS

    def kl(m1, s1, m2, s2):
        return jnp.log(s2 / s1) + (s1 ** 2 + (m1 - m2) ** 2) / (2.0 * s2 ** 2) - 0.5

    kl_ref[...] = kl(ml_t, msig_t, ml_c, msig_c) + kl(pmu_t, psig_t, pmu_c, psig_c)
    prob_ref[...] = prob_t
    rep_ref[...] = rep_t
    scale_ref[...] = scale_t
    tya = s_t[:, wt:wt + 1] / cnt_t
    ps = scale_t * prob_t
    g = ps.shape[0]
    tf_ref[...] = jnp.concatenate(
        [ps, tya, jnp.zeros((g, 3), F32)], axis=1)


def _run_p3(s_c, s_t, r_c, r_t, we2, be2, wm, bm, wms, bms, ws, bs,
            wpm, bpm, wps, bps, g, h, k):
    full = lambda arr: pl.BlockSpec(arr.shape, lambda: (0,) * arr.ndim)
    ins = [s_c, s_t, r_c, r_t, we2, be2, wm, bm, wms, bms, ws, bs,
           wpm, bpm, wps, bps]
    return pl.pallas_call(
        lambda *a: _p3_body(h, k, *a),
        in_specs=[full(a) for a in ins],
        out_specs=[
            pl.BlockSpec((g, k), lambda: (0, 0)),
            pl.BlockSpec((g, h), lambda: (0, 0)),
            pl.BlockSpec((g, k), lambda: (0, 0)),
            pl.BlockSpec((g, k), lambda: (0, 0)),
            pl.BlockSpec((g, k + 4), lambda: (0, 0)),
        ],
        out_shape=[
            jax.ShapeDtypeStruct((g, k), F32),
            jax.ShapeDtypeStruct((g, h), F32),
            jax.ShapeDtypeStruct((g, k), F32),
            jax.ShapeDtypeStruct((g, k), F32),
            jax.ShapeDtypeStruct((g, k + 4), F32),
        ],
    )(*ins)


def _p4_body(g, k, idx_ref, pred_ref, tf_ref, yp_ref, ys_ref):
    idx = idx_ref[0, 0, :]
    oh = _onehot(idx, g, BF16)
    gt = _gather_dot(oh, tf_ref[...])
    pred = pred_ref[...]
    mean = pred[:, :k]
    sig = _softplus(pred[:, k:2 * k]) * (1.0 - EPS) + EPS
    ps = gt[:, :k]
    tya = gt[:, k:k + 1]
    yp_ref[...] = jnp.sum(ps * mean, axis=1, keepdims=True) + tya
    ys_ref[...] = jnp.sqrt(jnp.sum((ps * sig) ** 2, axis=1, keepdims=True))


def _run_p4(idx3, pred, tf, g, k):
    n = pred.shape[0]
    nb = n // BN
    return pl.pallas_call(
        lambda *a: _p4_body(g, k, *a),
        grid=(nb,),
        in_specs=[
            pl.BlockSpec((1, 1, BN), lambda i: (i, 0, 0)),
            pl.BlockSpec((BN, 2 * k), lambda i: (i, 0)),
            pl.BlockSpec((g, k + 4), lambda i: (0, 0)),
        ],
        out_specs=[
            pl.BlockSpec((BN, 1), lambda i: (i, 0)),
            pl.BlockSpec((BN, 1), lambda i: (i, 0)),
        ],
        out_shape=[
            jax.ShapeDtypeStruct((n, 1), F32),
            jax.ShapeDtypeStruct((n, 1), F32),
        ],
        compiler_params=pltpu.CompilerParams(
            dimension_semantics=("arbitrary",)),
    )(idx3, pred, tf)


def kernel(context_index, context_x, context_y, target_index, target_x,
           target_y, We1, be1, We2, be2, Wm, bm, Wms, bms, Ws, bs, Wpm,
           bpm, Wps, bps, Wd1, bd1, Wd2, bd2):
    n, d = context_x.shape
    h = We2.shape[0]
    k = Wm.shape[1]
    g = 512
    nb = n // BN

    wx = We1[:d]
    wy = We1[d:d + 1]
    w1t = jnp.concatenate([wx, Wd1], axis=1)
    be1_2 = be1[None, :]
    bd1_2 = bd1[None, :]
    bd2_2 = bd2[None, :]

    cidx3 = context_index.astype(jnp.int32).reshape(nb, 1, BN)
    tidx3 = target_index.astype(jnp.int32).reshape(nb, 1, BN)

    ac, s_c = _run_p1(cidx3, context_x, context_y, wx, g)
    at, s_t = _run_p1(tidx3, target_x, target_y, w1t, g)

    (r_c,) = _run_p2(cidx3, ac, context_y, s_c, wy, be1_2, g, h, False)
    r_t, pred = _run_p2(tidx3, at, target_y, s_t, wy, be1_2, g, h, True,
                        bd1_2, Wd2, bd2_2)

    t_prob, t_rep, dist_kl, t_scale, tf = _run_p3(
        s_c, s_t, r_c, r_t, We2, be2[None, :], Wm, bm[None, :], Wms,
        bms[None, :], Ws, bs[None, :], Wpm, bpm[None, :], Wps,
        bps[None, :], g, h, k)

    y_pred, ys = _run_p4(tidx3, pred, tf, g, k)
    return (y_pred, ys[:, 0], t_prob, t_rep, dist_kl, t_scale)


# speedup vs baseline: 3.7631x; 1.0197x over previous
import jax
import jax.numpy as jnp
from jax import lax
from jax.experimental import pallas as pl
from jax.experimental.pallas import tpu as pltpu

F32 = jnp.float32
EPS = 0.01
BN = 2048


def _onehot(idx, g, dtype):
    iota = lax.broadcasted_iota(jnp.int32, (idx.shape[0], g), 1)
    return (iota == idx[:, None]).astype(dtype)


def _softplus(x):
    return jnp.maximum(x, 0.0) + jnp.log1p(jnp.exp(-jnp.abs(x)))


BF16 = jnp.bfloat16


def _split(b):
    hi = b.astype(BF16)
    lo = (b - hi.astype(F32)).astype(BF16)
    return hi, lo


def _seg_dot(oh, b):
    dn = (((0,), (0,)), ((), ()))
    hi, lo = _split(b)
    return (lax.dot_general(oh, hi, dn, preferred_element_type=F32) +
            lax.dot_general(oh, lo, dn, preferred_element_type=F32))


def _dot3(x, w):
    xh, xl = _split(x)
    wh, wl = _split(w)
    r = jnp.dot(xh, wh, preferred_element_type=F32)
    r += jnp.dot(xh, wl, preferred_element_type=F32)
    r += jnp.dot(xl, wh, preferred_element_type=F32)
    return r


def _gather_dot(oh, t):
    hi, lo = _split(t)
    return (jnp.dot(oh, hi, preferred_element_type=F32) +
            jnp.dot(oh, lo, preferred_element_type=F32))


def _p1_body(g, idx_ref, x_ref, y_ref, w_ref, a_ref, s_ref, oh_ref):
    i = pl.program_id(0)
    a = _dot3(x_ref[...], w_ref[...])
    a_ref[...] = a
    idx = idx_ref[0, 0, :]
    oh = _onehot(idx, g, BF16)
    oh_ref[...] = oh
    ones = jnp.ones((a.shape[0], 1), F32)
    b = jnp.concatenate([a, y_ref[...], ones], axis=1)
    contrib = _seg_dot(oh, b)

    @pl.when(i == 0)
    def _():
        s_ref[...] = jnp.zeros_like(s_ref)

    s_ref[...] += contrib


def _run_p1(idx3, x, y, w, g):
    n, d = x.shape
    wout = w.shape[1]
    nb = n // BN
    return pl.pallas_call(
        lambda *a: _p1_body(g, *a),
        grid=(nb,),
        in_specs=[
            pl.BlockSpec((1, 1, BN), lambda i: (i, 0, 0)),
            pl.BlockSpec((BN, d), lambda i: (i, 0)),
            pl.BlockSpec((BN, 1), lambda i: (i, 0)),
            pl.BlockSpec((d, wout), lambda i: (0, 0)),
        ],
        out_specs=[
            pl.BlockSpec((BN, wout), lambda i: (i, 0)),
            pl.BlockSpec((g, wout + 2), lambda i: (0, 0)),
            pl.BlockSpec((BN, g), lambda i: (i, 0)),
        ],
        out_shape=[
            jax.ShapeDtypeStruct((n, wout), F32),
            jax.ShapeDtypeStruct((g, wout + 2), F32),
            jax.ShapeDtypeStruct((n, g), BF16),
        ],
        compiler_params=pltpu.CompilerParams(
            dimension_semantics=("arbitrary",)),
    )(idx3, x, y, w)


def _p2_body(g, h, has_dec, *args):
    if has_dec:
        (oh_ref, a_ref, y_ref, s_ref, wy_ref, be1_ref, bd1_ref, wd2_ref,
         bd2_ref, r_ref, pred_ref, tt_ref) = args
    else:
        (oh_ref, a_ref, y_ref, s_ref, wy_ref, be1_ref, r_ref,
         tt_ref) = args
    i = pl.program_id(0)
    wa = a_ref.shape[1]

    @pl.when(i == 0)
    def _():
        s = s_ref[...]
        cnt = jnp.maximum(s[:, wa + 1:wa + 2], 1.0)
        te = s[:, :h] / cnt + (s[:, wa:wa + 1] / cnt) * wy_ref[...] \
            - be1_ref[...]
        if has_dec:
            td = s[:, h:wa] / cnt - bd1_ref[...]
            tt_ref[...] = jnp.concatenate([te, td], axis=1)
        else:
            tt_ref[...] = te
        r_ref[...] = jnp.zeros_like(r_ref)

    oh = oh_ref[...]
    gt = _gather_dot(oh, tt_ref[...])
    a = a_ref[...]
    h1 = jnp.tanh(a[:, :h] + y_ref[...] * wy_ref[...] - gt[:, :h])
    r_ref[...] += _seg_dot(oh, h1)
    if has_dec:
        d1 = jnp.tanh(a[:, h:] - gt[:, h:])
        pred_ref[...] = _dot3(d1, wd2_ref[...]) + bd2_ref[...]


def _run_p2(oh, a_stash, y, s, wy, be1, g, h, has_dec,
            bd1=None, wd2=None, bd2=None):
    n, wa = a_stash.shape
    nb = n // BN
    in_specs = [
        pl.BlockSpec((BN, g), lambda i: (i, 0)),
        pl.BlockSpec((BN, wa), lambda i: (i, 0)),
        pl.BlockSpec((BN, 1), lambda i: (i, 0)),
        pl.BlockSpec((g, wa + 2), lambda i: (0, 0)),
        pl.BlockSpec((1, h), lambda i: (0, 0)),
        pl.BlockSpec((1, h), lambda i: (0, 0)),
    ]
    outs = [pl.BlockSpec((g, h), lambda i: (0, 0))]
    out_shape = [jax.ShapeDtypeStruct((g, h), F32)]
    operands = [oh, a_stash, y, s, wy, be1]
    if has_dec:
        kout = wd2.shape[1]
        in_specs += [
            pl.BlockSpec((1, h), lambda i: (0, 0)),
            pl.BlockSpec((h, kout), lambda i: (0, 0)),
            pl.BlockSpec((1, kout), lambda i: (0, 0)),
        ]
        outs.append(pl.BlockSpec((BN, kout), lambda i: (i, 0)))
        out_shape.append(jax.ShapeDtypeStruct((n, kout), F32))
        operands += [bd1, wd2, bd2]
    return pl.pallas_call(
        lambda *args: _p2_body(g, h, has_dec, *args),
        grid=(nb,),
        in_specs=in_specs,
        out_specs=outs,
        out_shape=out_shape,
        scratch_shapes=[pltpu.VMEM((g, wa), F32)],
        compiler_params=pltpu.CompilerParams(
            dimension_semantics=("arbitrary",)),
    )(*operands)


def _p3_body(h, k, s_c_ref, s_t_ref, r_c_ref, r_t_ref, we2_ref, be2_ref,
             wm_ref, bm_ref, wms_ref, bms_ref, ws_ref, bs_ref, wpm_ref,
             bpm_ref, wps_ref, bps_ref,
             prob_ref, rep_ref, kl_ref, scale_ref, tf_ref):
    s_c = s_c_ref[...]
    s_t = s_t_ref[...]
    wc = s_c.shape[1] - 2
    wt = s_t.shape[1] - 2
    cnt_c = jnp.maximum(s_c[:, wc + 1:wc + 2], 1.0)
    cnt_t = jnp.maximum(s_t[:, wt + 1:wt + 2], 1.0)

    def mm(x, w_ref, b_ref):
        return jnp.dot(x, w_ref[...], preferred_element_type=F32,
                       precision=lax.Precision.HIGHEST) + b_ref[...]

    rep_c = mm(r_c_ref[...] / cnt_c, we2_ref, be2_ref)
    rep_t = mm(r_t_ref[...] / cnt_t, we2_ref, be2_ref)

    def heads(rep):
        ml = mm(rep, wm_ref, bm_ref)
        msig = _softplus(mm(rep, wms_ref, bms_ref)) * (1.0 - EPS) + EPS
        pmu = mm(rep, wpm_ref, bpm_ref)
        psig = _softplus(mm(rep, wps_ref, bps_ref)) * (1.0 - EPS) + EPS
        return ml, msig, pmu, psig

    ml_c, msig_c, pmu_c, psig_c = heads(rep_c)
    ml_t, msig_t, pmu_t, psig_t = heads(rep_t)

    mx = jnp.max(ml_t, axis=1, keepdims=True)
    ex = jnp.exp(ml_t - mx)
    prob_t = ex / jnp.sum(ex, axis=1, keepdims=True)
    scale_t = _softplus(mm(rep_t, ws_ref, bs_ref)) * (1.0 - EPS) + EPS

    def kl(m1, s1, m2, s2):
        return jnp.log(s2 / s1) + (s1 ** 2 + (m1 - m2) ** 2) / (2.0 * s2 ** 2) - 0.5

    kl_ref[...] = kl(ml_t, msig_t, ml_c, msig_c) + kl(pmu_t, psig_t, pmu_c, psig_c)
    prob_ref[...] = prob_t
    rep_ref[...] = rep_t
    scale_ref[...] = scale_t
    tya = s_t[:, wt:wt + 1] / cnt_t
    ps = scale_t * prob_t
    g = ps.shape[0]
    tf_ref[...] = jnp.concatenate(
        [ps, tya, jnp.zeros((g, 3), F32)], axis=1)


def _run_p3(s_c, s_t, r_c, r_t, we2, be2, wm, bm, wms, bms, ws, bs,
            wpm, bpm, wps, bps, g, h, k):
    full = lambda arr: pl.BlockSpec(arr.shape, lambda: (0,) * arr.ndim)
    ins = [s_c, s_t, r_c, r_t, we2, be2, wm, bm, wms, bms, ws, bs,
           wpm, bpm, wps, bps]
    return pl.pallas_call(
        lambda *a: _p3_body(h, k, *a),
        in_specs=[full(a) for a in ins],
        out_specs=[
            pl.BlockSpec((g, k), lambda: (0, 0)),
            pl.BlockSpec((g, h), lambda: (0, 0)),
            pl.BlockSpec((g, k), lambda: (0, 0)),
            pl.BlockSpec((g, k), lambda: (0, 0)),
            pl.BlockSpec((g, k + 4), lambda: (0, 0)),
        ],
        out_shape=[
            jax.ShapeDtypeStruct((g, k), F32),
            jax.ShapeDtypeStruct((g, h), F32),
            jax.ShapeDtypeStruct((g, k), F32),
            jax.ShapeDtypeStruct((g, k), F32),
            jax.ShapeDtypeStruct((g, k + 4), F32),
        ],
    )(*ins)


def _p4_body(g, k, oh_ref, pred_ref, tf_ref, yp_ref, ys_ref):
    oh = oh_ref[...]
    gt = _gather_dot(oh, tf_ref[...])
    pred = pred_ref[...]
    mean = pred[:, :k]
    sig = _softplus(pred[:, k:2 * k]) * (1.0 - EPS) + EPS
    ps = gt[:, :k]
    tya = gt[:, k:k + 1]
    yp_ref[...] = jnp.sum(ps * mean, axis=1, keepdims=True) + tya
    ys_ref[...] = jnp.sqrt(jnp.sum((ps * sig) ** 2, axis=1, keepdims=True))


def _run_p4(oh, pred, tf, g, k):
    n = pred.shape[0]
    nb = n // BN
    return pl.pallas_call(
        lambda *a: _p4_body(g, k, *a),
        grid=(nb,),
        in_specs=[
            pl.BlockSpec((BN, g), lambda i: (i, 0)),
            pl.BlockSpec((BN, 2 * k), lambda i: (i, 0)),
            pl.BlockSpec((g, k + 4), lambda i: (0, 0)),
        ],
        out_specs=[
            pl.BlockSpec((BN, 1), lambda i: (i, 0)),
            pl.BlockSpec((BN, 1), lambda i: (i, 0)),
        ],
        out_shape=[
            jax.ShapeDtypeStruct((n, 1), F32),
            jax.ShapeDtypeStruct((n, 1), F32),
        ],
        compiler_params=pltpu.CompilerParams(
            dimension_semantics=("arbitrary",)),
    )(oh, pred, tf)


def kernel(context_index, context_x, context_y, target_index, target_x,
           target_y, We1, be1, We2, be2, Wm, bm, Wms, bms, Ws, bs, Wpm,
           bpm, Wps, bps, Wd1, bd1, Wd2, bd2):
    n, d = context_x.shape
    h = We2.shape[0]
    k = Wm.shape[1]
    g = 512
    nb = n // BN

    wx = We1[:d]
    wy = We1[d:d + 1]
    w1t = jnp.concatenate([wx, Wd1], axis=1)
    be1_2 = be1[None, :]
    bd1_2 = bd1[None, :]
    bd2_2 = bd2[None, :]

    cidx3 = context_index.astype(jnp.int32).reshape(nb, 1, BN)
    tidx3 = target_index.astype(jnp.int32).reshape(nb, 1, BN)

    ac, s_c, oh_c = _run_p1(cidx3, context_x, context_y, wx, g)
    at, s_t, oh_t = _run_p1(tidx3, target_x, target_y, w1t, g)

    (r_c,) = _run_p2(oh_c, ac, context_y, s_c, wy, be1_2, g, h, False)
    r_t, pred = _run_p2(oh_t, at, target_y, s_t, wy, be1_2, g, h, True,
                        bd1_2, Wd2, bd2_2)

    t_prob, t_rep, dist_kl, t_scale, tf = _run_p3(
        s_c, s_t, r_c, r_t, We2, be2[None, :], Wm, bm[None, :], Wms,
        bms[None, :], Ws, bs[None, :], Wpm, bpm[None, :], Wps,
        bps[None, :], g, h, k)

    y_pred, ys = _run_p4(oh_t, pred, tf, g, k)
    return (y_pred, ys[:, 0], t_prob, t_rep, dist_kl, t_scale)


# speedup vs baseline: 4.9468x; 1.3146x over previous
import jax
import jax.numpy as jnp
from jax import lax
from jax.experimental import pallas as pl
from jax.experimental.pallas import tpu as pltpu

F32 = jnp.float32
EPS = 0.01
BN = 2048
GB = 128


def _onehot_band(idx, lo, gb, dtype):
    iota = lax.broadcasted_iota(jnp.int32, (idx.shape[0], gb), 1)
    return (iota == (idx[:, None] - lo)).astype(dtype)


def _softplus(x):
    return jnp.maximum(x, 0.0) + jnp.log1p(jnp.exp(-jnp.abs(x)))


BF16 = jnp.bfloat16


def _split(b):
    hi = b.astype(BF16)
    lo = (b - hi.astype(F32)).astype(BF16)
    return hi, lo


def _seg_dot2(oh, bh, bl):
    dn = (((0,), (0,)), ((), ()))
    return (lax.dot_general(oh, bh, dn, preferred_element_type=F32) +
            lax.dot_general(oh, bl, dn, preferred_element_type=F32))


def _dot3(x, w):
    xh, xl = _split(x)
    wh, wl = _split(w)
    r = jnp.dot(xh, wh, preferred_element_type=F32)
    r += jnp.dot(xh, wl, preferred_element_type=F32)
    r += jnp.dot(xl, wh, preferred_element_type=F32)
    return r


def _gather_dot2(oh, th, tl):
    return (jnp.dot(oh, th, preferred_element_type=F32) +
            jnp.dot(oh, tl, preferred_element_type=F32))


def _p1_body(g, idx_ref, x_ref, y_ref, w_ref, flg_ref, a_ref, s_ref):
    i = pl.program_id(0)
    a = _dot3(x_ref[...], w_ref[...])
    a_ref[...] = a
    idx = idx_ref[0, 0, :]
    ones = jnp.ones((a.shape[0], 1), F32)
    b = jnp.concatenate([a, y_ref[...], ones], axis=1)
    bh, bl = _split(b)

    @pl.when(i == 0)
    def _():
        s_ref[...] = jnp.zeros_like(s_ref)

    for j in range(g // GB):
        @pl.when(flg_ref[i, j] != 0)
        def _(j=j):
            oh = _onehot_band(idx, j * GB, GB, BF16)
            s_ref[j * GB:(j + 1) * GB, :] += _seg_dot2(oh, bh, bl)


def _run_p1(idx3, x, y, w, flg, g):
    n, d = x.shape
    wout = w.shape[1]
    nb = n // BN
    return pl.pallas_call(
        lambda *a: _p1_body(g, *a),
        grid=(nb,),
        in_specs=[
            pl.BlockSpec((1, 1, BN), lambda i: (i, 0, 0)),
            pl.BlockSpec((BN, d), lambda i: (i, 0)),
            pl.BlockSpec((BN, 1), lambda i: (i, 0)),
            pl.BlockSpec((d, wout), lambda i: (0, 0)),
            pl.BlockSpec((nb, g // GB), lambda i: (0, 0),
                         memory_space=pltpu.SMEM),
        ],
        out_specs=[
            pl.BlockSpec((BN, wout), lambda i: (i, 0)),
            pl.BlockSpec((g, wout + 2), lambda i: (0, 0)),
        ],
        out_shape=[
            jax.ShapeDtypeStruct((n, wout), F32),
            jax.ShapeDtypeStruct((g, wout + 2), F32),
        ],
        compiler_params=pltpu.CompilerParams(
            dimension_semantics=("arbitrary",)),
    )(idx3, x, y, w, flg)


def _p2_body(g, h, has_dec, *args):
    if has_dec:
        (idx_ref, a_ref, y_ref, s_ref, wy_ref, be1_ref, flg_ref, bd1_ref,
         wd2_ref, bd2_ref, r_ref, pred_ref, tt_ref, gt_ref) = args
    else:
        (idx_ref, a_ref, y_ref, s_ref, wy_ref, be1_ref, flg_ref, r_ref,
         tt_ref, gt_ref) = args
    i = pl.program_id(0)
    wa = a_ref.shape[1]

    @pl.when(i == 0)
    def _():
        s = s_ref[...]
        cnt = jnp.maximum(s[:, wa + 1:wa + 2], 1.0)
        te = s[:, :h] / cnt + (s[:, wa:wa + 1] / cnt) * wy_ref[...] \
            - be1_ref[...]
        if has_dec:
            td = s[:, h:wa] / cnt - bd1_ref[...]
            tt = jnp.concatenate([te, td], axis=1)
        else:
            tt = te
        th, tl = _split(tt)
        tt_ref[...] = jnp.concatenate([th, tl], axis=1)
        r_ref[...] = jnp.zeros_like(r_ref)

    idx = idx_ref[0, 0, :]
    gt_ref[...] = jnp.zeros_like(gt_ref)
    for j in range(g // GB):
        @pl.when(flg_ref[i, j] != 0)
        def _(j=j):
            oh = _onehot_band(idx, j * GB, GB, BF16)
            th = tt_ref[j * GB:(j + 1) * GB, :wa]
            tl = tt_ref[j * GB:(j + 1) * GB, wa:]
            gt_ref[...] += _gather_dot2(oh, th, tl)

    gt = gt_ref[...]
    a = a_ref[...]
    h1 = jnp.tanh(a[:, :h] + y_ref[...] * wy_ref[...] - gt[:, :h])
    hh, hl = _split(h1)
    for j in range(g // GB):
        @pl.when(flg_ref[i, j] != 0)
        def _(j=j):
            oh = _onehot_band(idx, j * GB, GB, BF16)
            r_ref[j * GB:(j + 1) * GB, :] += _seg_dot2(oh, hh, hl)
    if has_dec:
        d1 = jnp.tanh(a[:, h:] - gt[:, h:])
        pred_ref[...] = _dot3(d1, wd2_ref[...]) + bd2_ref[...]


def _run_p2(idx3, a_stash, y, s, wy, be1, flg, g, h, has_dec,
            bd1=None, wd2=None, bd2=None):
    n, wa = a_stash.shape
    nb = n // BN
    in_specs = [
        pl.BlockSpec((1, 1, BN), lambda i: (i, 0, 0)),
        pl.BlockSpec((BN, wa), lambda i: (i, 0)),
        pl.BlockSpec((BN, 1), lambda i: (i, 0)),
        pl.BlockSpec((g, wa + 2), lambda i: (0, 0)),
        pl.BlockSpec((1, h), lambda i: (0, 0)),
        pl.BlockSpec((1, h), lambda i: (0, 0)),
        pl.BlockSpec((nb, g // GB), lambda i: (0, 0),
                     memory_space=pltpu.SMEM),
    ]
    outs = [pl.BlockSpec((g, h), lambda i: (0, 0))]
    out_shape = [jax.ShapeDtypeStruct((g, h), F32)]
    operands = [idx3, a_stash, y, s, wy, be1, flg]
    if has_dec:
        kout = wd2.shape[1]
        in_specs += [
            pl.BlockSpec((1, h), lambda i: (0, 0)),
            pl.BlockSpec((h, kout), lambda i: (0, 0)),
            pl.BlockSpec((1, kout), lambda i: (0, 0)),
        ]
        outs.append(pl.BlockSpec((BN, kout), lambda i: (i, 0)))
        out_shape.append(jax.ShapeDtypeStruct((n, kout), F32))
        operands += [bd1, wd2, bd2]
    return pl.pallas_call(
        lambda *args: _p2_body(g, h, has_dec, *args),
        grid=(nb,),
        in_specs=in_specs,
        out_specs=outs,
        out_shape=out_shape,
        scratch_shapes=[pltpu.VMEM((g, 2 * wa), BF16),
                        pltpu.VMEM((BN, wa), F32)],
        compiler_params=pltpu.CompilerParams(
            dimension_semantics=("arbitrary",)),
    )(*operands)


def _p3_body(h, k, s_c_ref, s_t_ref, r_c_ref, r_t_ref, we2_ref, be2_ref,
             wm_ref, bm_ref, wms_ref, bms_ref, ws_ref, bs_ref, wpm_ref,
             bpm_ref, wps_ref, bps_ref,
             prob_ref, rep_ref, kl_ref, scale_ref, tf_ref):
    s_c = s_c_ref[...]
    s_t = s_t_ref[...]
    wc = s_c.shape[1] - 2
    wt = s_t.shape[1] - 2
    cnt_c = jnp.maximum(s_c[:, wc + 1:wc + 2], 1.0)
    cnt_t = jnp.maximum(s_t[:, wt + 1:wt + 2], 1.0)

    def mm(x, w_ref, b_ref):
        return jnp.dot(x, w_ref[...], preferred_element_type=F32,
                       precision=lax.Precision.HIGHEST) + b_ref[...]

    rep_c = mm(r_c_ref[...] / cnt_c, we2_ref, be2_ref)
    rep_t = mm(r_t_ref[...] / cnt_t, we2_ref, be2_ref)

    def heads(rep):
        ml = mm(rep, wm_ref, bm_ref)
        msig = _softplus(mm(rep, wms_ref, bms_ref)) * (1.0 - EPS) + EPS
        pmu = mm(rep, wpm_ref, bpm_ref)
        psig = _softplus(mm(rep, wps_ref, bps_ref)) * (1.0 - EPS) + EPS
        return ml, msig, pmu, psig

    ml_c, msig_c, pmu_c, psig_c = heads(rep_c)
    ml_t, msig_t, pmu_t, psig_t = heads(rep_t)

    mx = jnp.max(ml_t, axis=1, keepdims=True)
    ex = jnp.exp(ml_t - mx)
    prob_t = ex / jnp.sum(ex, axis=1, keepdims=True)
    scale_t = _softplus(mm(rep_t, ws_ref, bs_ref)) * (1.0 - EPS) + EPS

    def kl(m1, s1, m2, s2):
        return jnp.log(s2 / s1) + (s1 ** 2 + (m1 - m2) ** 2) / (2.0 * s2 ** 2) - 0.5

    kl_ref[...] = kl(ml_t, msig_t, ml_c, msig_c) + kl(pmu_t, psig_t, pmu_c, psig_c)
    prob_ref[...] = prob_t
    rep_ref[...] = rep_t
    scale_ref[...] = scale_t
    tya = s_t[:, wt:wt + 1] / cnt_t
    ps = scale_t * prob_t
    g = ps.shape[0]
    tf_ref[...] = jnp.concatenate(
        [ps, tya, jnp.zeros((g, 3), F32)], axis=1)


def _run_p3(s_c, s_t, r_c, r_t, we2, be2, wm, bm, wms, bms, ws, bs,
            wpm, bpm, wps, bps, g, h, k):
    full = lambda arr: pl.BlockSpec(arr.shape, lambda: (0,) * arr.ndim)
    ins = [s_c, s_t, r_c, r_t, we2, be2, wm, bm, wms, bms, ws, bs,
           wpm, bpm, wps, bps]
    return pl.pallas_call(
        lambda *a: _p3_body(h, k, *a),
        in_specs=[full(a) for a in ins],
        out_specs=[
            pl.BlockSpec((g, k), lambda: (0, 0)),
            pl.BlockSpec((g, h), lambda: (0, 0)),
            pl.BlockSpec((g, k), lambda: (0, 0)),
            pl.BlockSpec((g, k), lambda: (0, 0)),
            pl.BlockSpec((g, k + 4), lambda: (0, 0)),
        ],
        out_shape=[
            jax.ShapeDtypeStruct((g, k), F32),
            jax.ShapeDtypeStruct((g, h), F32),
            jax.ShapeDtypeStruct((g, k), F32),
            jax.ShapeDtypeStruct((g, k), F32),
            jax.ShapeDtypeStruct((g, k + 4), F32),
        ],
    )(*ins)


def _p4_body(g, k, idx_ref, pred_ref, tf_ref, yp_ref, ys_ref):
    idx = idx_ref[0, 0, :]
    oh = _onehot_band(idx, 0, g, BF16)
    tf = tf_ref[...]
    th, tl = _split(tf)
    gt = _gather_dot2(oh, th, tl)
    pred = pred_ref[...]
    mean = pred[:, :k]
    sig = _softplus(pred[:, k:2 * k]) * (1.0 - EPS) + EPS
    ps = gt[:, :k]
    tya = gt[:, k:k + 1]
    yp_ref[...] = jnp.sum(ps * mean, axis=1, keepdims=True) + tya
    ys_ref[...] = jnp.sqrt(jnp.sum((ps * sig) ** 2, axis=1, keepdims=True))


def _run_p4(idx3, pred, tf, g, k):
    n = pred.shape[0]
    nb = n // BN
    return pl.pallas_call(
        lambda *a: _p4_body(g, k, *a),
        grid=(nb,),
        in_specs=[
            pl.BlockSpec((1, 1, BN), lambda i: (i, 0, 0)),
            pl.BlockSpec((BN, 2 * k), lambda i: (i, 0)),
            pl.BlockSpec((g, k + 4), lambda i: (0, 0)),
        ],
        out_specs=[
            pl.BlockSpec((BN, 1), lambda i: (i, 0)),
            pl.BlockSpec((BN, 1), lambda i: (i, 0)),
        ],
        out_shape=[
            jax.ShapeDtypeStruct((n, 1), F32),
            jax.ShapeDtypeStruct((n, 1), F32),
        ],
        compiler_params=pltpu.CompilerParams(
            dimension_semantics=("arbitrary",)),
    )(idx3, pred, tf)


def _chunk_flags(idx, nb, g):
    ii = idx.astype(jnp.int32).reshape(nb, BN)
    bmin = ii.min(axis=1)
    bmax = ii.max(axis=1)
    jlo = jnp.arange(g // GB, dtype=jnp.int32) * GB
    jhi = jlo + (GB - 1)
    return ((bmin[:, None] <= jhi[None, :]) &
            (bmax[:, None] >= jlo[None, :])).astype(jnp.int32)


def kernel(context_index, context_x, context_y, target_index, target_x,
           target_y, We1, be1, We2, be2, Wm, bm, Wms, bms, Ws, bs, Wpm,
           bpm, Wps, bps, Wd1, bd1, Wd2, bd2):
    n, d = context_x.shape
    h = We2.shape[0]
    k = Wm.shape[1]
    g = 512
    nb = n // BN

    wx = We1[:d]
    wy = We1[d:d + 1]
    w1t = jnp.concatenate([wx, Wd1], axis=1)
    be1_2 = be1[None, :]
    bd1_2 = bd1[None, :]
    bd2_2 = bd2[None, :]

    cidx3 = context_index.astype(jnp.int32).reshape(nb, 1, BN)
    tidx3 = target_index.astype(jnp.int32).reshape(nb, 1, BN)
    cflg = _chunk_flags(context_index, nb, g)
    tflg = _chunk_flags(target_index, nb, g)

    ac, s_c = _run_p1(cidx3, context_x, context_y, wx, cflg, g)
    at, s_t = _run_p1(tidx3, target_x, target_y, w1t, tflg, g)

    (r_c,) = _run_p2(cidx3, ac, context_y, s_c, wy, be1_2, cflg, g, h,
                     False)
    r_t, pred = _run_p2(tidx3, at, target_y, s_t, wy, be1_2, tflg, g, h,
                        True, bd1_2, Wd2, bd2_2)

    t_prob, t_rep, dist_kl, t_scale, tf = _run_p3(
        s_c, s_t, r_c, r_t, We2, be2[None, :], Wm, bm[None, :], Wms,
        bms[None, :], Ws, bs[None, :], Wpm, bpm[None, :], Wps,
        bps[None, :], g, h, k)

    y_pred, ys = _run_p4(tidx3, pred, tf, g, k)
    return (y_pred, ys[:, 0], t_prob, t_rep, dist_kl, t_scale)
